# Initial kernel scaffold; baseline (speedup 1.0000x reference)
import jax
import jax.numpy as jnp
from jax.experimental import pallas as pl


def kernel(x, edge_index, W1, b1, g1, be1, W2, b2, g2, be2, W3, b3):
    raise NotImplementedError("write your pallas kernel here")


# speedup vs baseline: 11.9979x; 11.9979x over previous
import functools

import jax
import jax.numpy as jnp
from jax import lax
from jax.experimental import pallas as pl
from jax.experimental.pallas import tpu as pltpu
from jax.experimental.pallas import tpu_sc as plsc

N = 10000
E = 320000
F_IN = 128
H = 128
C = 40
EPS = 1e-5

NC = 2
NS = 16
NW = NC * NS
CHUNK = 128
K = -(-E // (NW * CHUNK))
E_PAD = NW * K * CHUNK
ZROW = N
N_PAD = 10240
RPT = N_PAD // NS
C_PAD = 48
BLK = 1024

_mesh = lambda: plsc.VectorSubcoreMesh(core_axis_name="c", subcore_axis_name="s")


def _sc_deg(dst_slab, zvec):
    @functools.partial(
        pl.kernel,
        mesh=_mesh(),
        out_type=jax.ShapeDtypeStruct((NW, N_PAD), jnp.float32),
        scratch_types=[
            pltpu.VMEM((K, CHUNK), jnp.int32),
            pltpu.VMEM((N_PAD,), jnp.float32),
        ],
        compiler_params=pltpu.CompilerParams(needs_layout_passes=False),
    )
    def k(dst_hbm, z_hbm, out_hbm, dst_v, hist_v):
        c = lax.axis_index("c")
        s = lax.axis_index("s")
        wid = s * NC + c
        pltpu.sync_copy(dst_hbm.at[wid], dst_v)
        pltpu.sync_copy(z_hbm, hist_v)
        ones = jnp.ones((16,), jnp.float32)

        def body(j, carry):
            for l in range(CHUNK // 16):
                idx = dst_v[j, pl.ds(l * 16, 16)]
                plsc.addupdate_scatter(hist_v, [idx], ones)
            return carry

        lax.fori_loop(0, K, body, 0)
        pltpu.sync_copy(hist_v, out_hbm.at[wid])

    return k(dst_slab, zvec)


def _sc_agg(h, src_slab, dst_slab, zrows, feat):
    @functools.partial(
        pl.kernel,
        mesh=_mesh(),
        out_type=jax.ShapeDtypeStruct((NC, N_PAD, feat), jnp.float32),
        scratch_types=[
            pltpu.VMEM((K, CHUNK), jnp.int32),
            pltpu.VMEM((K, CHUNK), jnp.int32),
            pltpu.VMEM((CHUNK, feat), jnp.float32),
            pltpu.VMEM_SHARED((N_PAD, feat), jnp.float32),
            pltpu.SemaphoreType.DMA,
        ],
        compiler_params=pltpu.CompilerParams(use_tc_tiling_on_sc=False),
    )
    def k(h_hbm, src_hbm, dst_hbm, z_hbm, out_hbm, src_v, dst_v, buf, acc, sem):
        c = lax.axis_index("c")
        s = lax.axis_index("s")
        wid = s * NC + c
        pltpu.sync_copy(src_hbm.at[wid], src_v)
        pltpu.sync_copy(dst_hbm.at[wid], dst_v)
        pltpu.sync_copy(z_hbm.at[pl.ds(s * RPT, RPT)], acc.at[pl.ds(s * RPT, RPT)])
        plsc.subcore_barrier()

        def body(j, carry):
            pltpu.async_copy(h_hbm.at[src_v.at[j]], buf, sem).wait()
            pltpu.sync_copy(buf, acc.at[dst_v.at[j]], add=True)
            return carry

        lax.fori_loop(0, K, body, 0)
        plsc.subcore_barrier()
        pltpu.sync_copy(acc.at[pl.ds(s * RPT, RPT)],
                        out_hbm.at[c, pl.ds(s * RPT, RPT)])

    return k(h, src_slab, dst_slab, zrows)


def _tc1(deg_t, x_pad, W1):
    def body(degp, xb, w, dinv_o, u_o):
        deg = jnp.sum(degp[...], axis=1, keepdims=True) + 1.0
        dv = lax.rsqrt(deg)
        dinv_o[...] = dv
        u = jnp.dot(xb[...], w[...], preferred_element_type=jnp.float32)
        u_o[...] = u * dv

    grid = (N_PAD // BLK,)
    return pl.pallas_call(
        body,
        grid=grid,
        in_specs=[
            pl.BlockSpec((BLK, NW), lambda i: (i, 0)),
            pl.BlockSpec((BLK, F_IN), lambda i: (i, 0)),
            pl.BlockSpec((F_IN, H), lambda i: (0, 0)),
        ],
        out_specs=[
            pl.BlockSpec((BLK, 1), lambda i: (i, 0)),
            pl.BlockSpec((BLK, H), lambda i: (i, 0)),
        ],
        out_shape=[
            jax.ShapeDtypeStruct((N_PAD, 1), jnp.float32),
            jax.ShapeDtypeStruct((N_PAD, H), jnp.float32),
        ],
    )(deg_t, x_pad, W1)


def _tc_layer(p0, p1, us, dinv, b, g, be, W_next, x_res, feat_out):
    feat_in = us.shape[1]
    bnscale = 1.0 / (1.0 + EPS) ** 0.5

    def body(*refs):
        if x_res is not None:
            p0r, p1r, usr, dvr, br, gr, ber, wr, xr, out_o = refs
        else:
            p0r, p1r, usr, dvr, br, gr, ber, wr, out_o = refs
        i = pl.program_id(0)
        dv = dvr[...]
        conv = (p0r[...] + p1r[...] + usr[...]) * dv + br[...]
        hcur = conv * (gr[...] * bnscale) + ber[...]
        if x_res is not None:
            hcur = hcur + xr[...]
        hcur = jnp.maximum(hcur, 0.0)
        u = jnp.dot(hcur, wr[...], preferred_element_type=jnp.float32)
        rowid = lax.broadcasted_iota(jnp.int32, (BLK, 1), 0) + i * BLK
        out_o[...] = jnp.where(rowid < N, u * dv, 0.0)

    grid = (N_PAD // BLK,)
    fspec = pl.BlockSpec((BLK, feat_in), lambda i: (i, 0))
    vspec = pl.BlockSpec((1, feat_in), lambda i: (0, 0))
    in_specs = [fspec, fspec, fspec,
                pl.BlockSpec((BLK, 1), lambda i: (i, 0)),
                vspec, vspec, vspec,
                pl.BlockSpec((feat_in, feat_out), lambda i: (0, 0))]
    args = [p0, p1, us, dinv, b, g, be, W_next]
    if x_res is not None:
        in_specs.append(fspec)
        args.append(x_res)
    return pl.pallas_call(
        body,
        grid=grid,
        in_specs=in_specs,
        out_specs=pl.BlockSpec((BLK, feat_out), lambda i: (i, 0)),
        out_shape=jax.ShapeDtypeStruct((N_PAD, feat_out), jnp.float32),
    )(*args)


def _tc_final(p0, p1, us, dinv, b3p):
    def body(p0r, p1r, usr, dvr, br, out_o):
        conv = (p0r[...] + p1r[...] + usr[...]) * dvr[...] + br[...]
        colid = lax.broadcasted_iota(jnp.int32, (BLK, C_PAD), 1)
        valid = colid < C
        neg = jnp.float32(-1e30)
        mx = jnp.max(jnp.where(valid, conv, neg), axis=1, keepdims=True)
        z = conv - mx
        ssum = jnp.sum(jnp.where(valid, jnp.exp(z), 0.0), axis=1, keepdims=True)
        out_o[...] = z - jnp.log(ssum)

    grid = (N_PAD // BLK,)
    fspec = pl.BlockSpec((BLK, C_PAD), lambda i: (i, 0))
    return pl.pallas_call(
        body,
        grid=grid,
        in_specs=[fspec, fspec, fspec,
                  pl.BlockSpec((BLK, 1), lambda i: (i, 0)),
                  pl.BlockSpec((1, C_PAD), lambda i: (0, 0))],
        out_specs=fspec,
        out_shape=jax.ShapeDtypeStruct((N_PAD, C_PAD), jnp.float32),
    )(p0, p1, us, dinv, b3p)


def kernel(x, edge_index, W1, b1, g1, be1, W2, b2, g2, be2, W3, b3):
    padi = jnp.full((E_PAD - E,), ZROW, jnp.int32)
    src_slab = jnp.concatenate([edge_index[0], padi]).reshape(NW, K, CHUNK)
    dst_slab = jnp.concatenate([edge_index[1], padi]).reshape(NW, K, CHUNK)
    x_pad = jnp.zeros((N_PAD, F_IN), jnp.float32).at[:N].set(x)
    zvec = jnp.zeros((N_PAD,), jnp.float32)
    zrows_h = jnp.zeros((N_PAD, H), jnp.float32)
    zrows_c = jnp.zeros((N_PAD, C_PAD), jnp.float32)
    W3p = jnp.zeros((H, C_PAD), jnp.float32).at[:, :C].set(W3)
    b1r = b1.reshape(1, H)
    g1r = g1.reshape(1, H)
    be1r = be1.reshape(1, H)
    b2r = b2.reshape(1, H)
    g2r = g2.reshape(1, H)
    be2r = be2.reshape(1, H)
    b3p = jnp.zeros((1, C_PAD), jnp.float32).at[0, :C].set(b3)

    deg_parts = _sc_deg(dst_slab, zvec)
    deg_t = deg_parts.T
    dinv, u1s = _tc1(deg_t, x_pad, W1)

    parts = _sc_agg(u1s, src_slab, dst_slab, zrows_h, H)
    u2s = _tc_layer(parts[0], parts[1], u1s, dinv, b1r, g1r, be1r, W2,
                    x_pad, H)
    parts = _sc_agg(u2s, src_slab, dst_slab, zrows_h, H)
    u3s = _tc_layer(parts[0], parts[1], u2s, dinv, b2r, g2r, be2r, W3p,
                    None, C_PAD)
    parts = _sc_agg(u3s, src_slab, dst_slab, zrows_c, C_PAD)
    out = _tc_final(parts[0], parts[1], u3s, dinv, b3p)
    return out[:N, :C]
